# baseline (speedup 1.0000x reference)
import jax
import jax.numpy as jnp
import numpy as np
from jax.experimental import pallas as pl

TOP_K = 30
NUM_RBF = 16
NUM_PE = 16


def _l2norm(v, axis=-1, eps=1e-12):
    n = jnp.sqrt(jnp.sum(v * v, axis=axis, keepdims=True))
    return v / jnp.maximum(n, eps)


def _gather_nodes(nodes, neighbor_idx):
    B, N, K = neighbor_idx.shape
    idx = neighbor_idx.reshape(B, N * K)[:, :, None]
    out = jnp.take_along_axis(nodes, idx, axis=1)
    return out.reshape(B, N, K, nodes.shape[-1])


def _dist_jax(x, mask, eps=1e-6):
    mask_2d = mask[:, None, :] * mask[:, :, None]
    dX = x[:, None, :, :] - x[:, :, None, :]
    D = mask_2d * jnp.sqrt(jnp.sum(dX ** 2, axis=3) + eps)
    D_max = jnp.max(D, axis=-1, keepdims=True)
    D_adjust = D + (1.0 - mask_2d) * D_max
    neg_vals, edge_idx = jax.lax.top_k(-D_adjust, TOP_K)
    return -neg_vals, edge_idx


def _rbf(D):
    D_mu = jnp.linspace(0.0, 20.0, NUM_RBF).reshape(1, 1, 1, -1)
    D_sigma = 20.0 / NUM_RBF
    return jnp.exp(-(((D[..., None] - D_mu) / D_sigma) ** 2))


def _quaternions(R):
    diag = jnp.diagonal(R, axis1=-2, axis2=-1)
    Rxx, Ryy, Rzz = diag[..., 0], diag[..., 1], diag[..., 2]
    magnitudes = 0.5 * jnp.sqrt(jnp.abs(1 + jnp.stack([Rxx - Ryy - Rzz, -Rxx + Ryy - Rzz, -Rxx - Ryy + Rzz], axis=-1)))
    signs = jnp.sign(jnp.stack([R[..., 2, 1] - R[..., 1, 2], R[..., 0, 2] - R[..., 2, 0], R[..., 1, 0] - R[..., 0, 1]], axis=-1))
    xyz = signs * magnitudes
    w = jnp.sqrt(jax.nn.relu(1 + jnp.sum(diag, axis=-1, keepdims=True))) / 2.0
    Q = jnp.concatenate([xyz, w], axis=-1)
    return _l2norm(Q)


def _orientations(x, edge_idx):
    dX = x[:, 1:, :] - x[:, :-1, :]
    U = _l2norm(dX)
    u_2 = U[:, :-2, :]
    u_1 = U[:, 1:-1, :]
    n_2 = _l2norm(jnp.cross(u_2, u_1))
    o_1 = _l2norm(u_2 - u_1)
    O = jnp.stack([o_1, n_2, jnp.cross(o_1, n_2)], axis=2)
    O = O.reshape(O.shape[0], O.shape[1], 9)
    O = jnp.pad(O, ((0, 0), (1, 2), (0, 0)))
    O_neighbors = _gather_nodes(O, edge_idx)
    X_neighbors = _gather_nodes(x, edge_idx)
    B, N = O.shape[0], O.shape[1]
    K = edge_idx.shape[2]
    O = O.reshape(B, N, 3, 3)
    O_neighbors = O_neighbors.reshape(B, N, K, 3, 3)
    dXn = X_neighbors - x[:, :, None, :]
    dU = jnp.matmul(O[:, :, None], dXn[..., None])[..., 0]
    dU = _l2norm(dU)
    R = jnp.matmul(jnp.swapaxes(O[:, :, None], -1, -2), O_neighbors)
    Q = _quaternions(R)
    return jnp.concatenate([dU, Q], axis=-1)


def _pe(edge_idx):
    n_nodes = edge_idx.shape[1]
    ii = jnp.arange(n_nodes, dtype=jnp.float32).reshape(1, -1, 1)
    d = (edge_idx.astype(jnp.float32) - ii)[..., None]
    frequency = jnp.exp(jnp.arange(0, NUM_PE, 2, dtype=jnp.float32) * (-(np.log(10000.0) / NUM_PE)))
    angles = d * frequency.reshape(1, 1, 1, -1)
    return jnp.concatenate([jnp.cos(angles), jnp.sin(angles)], axis=-1)


def _linear_ln_body(e_ref, wt_ref, b_ref, g_ref, beta_ref, o_ref):
    e = e_ref[...]
    y = jnp.dot(e, wt_ref[...], preferred_element_type=jnp.float32) + b_ref[...]
    mu = jnp.mean(y, axis=-1, keepdims=True)
    d = y - mu
    var = jnp.sum(d * d, axis=-1, keepdims=True) * (1.0 / (y.shape[-1] - 1))
    sigma = jnp.sqrt(var + 1e-6)
    o_ref[...] = g_ref[...] * d / (sigma + 1e-6) + beta_ref[...]


def _linear_ln(E_feat, W, b, gain, bias):
    M, F = E_feat.shape
    OUT = W.shape[0]
    BM = 2048
    grid = (M // BM,)
    return pl.pallas_call(
        _linear_ln_body,
        grid=grid,
        in_specs=[
            pl.BlockSpec((BM, F), lambda i: (i, 0)),
            pl.BlockSpec((F, OUT), lambda i: (0, 0)),
            pl.BlockSpec((1, OUT), lambda i: (0, 0)),
            pl.BlockSpec((1, OUT), lambda i: (0, 0)),
            pl.BlockSpec((1, OUT), lambda i: (0, 0)),
        ],
        out_specs=pl.BlockSpec((BM, OUT), lambda i: (i, 0)),
        out_shape=jax.ShapeDtypeStruct((M, OUT), jnp.float32),
    )(E_feat, W.T, b.reshape(1, OUT), gain.reshape(1, OUT), bias.reshape(1, OUT))


def kernel(x, mask, W, b, gain, bias):
    D_neighbors, edge_idx = _dist_jax(x, mask)
    rbf = _rbf(D_neighbors)
    o_features = _orientations(x, edge_idx)
    e_positional = _pe(edge_idx)
    E = jnp.concatenate([e_positional, rbf, o_features], axis=-1)
    B, N, K, F = E.shape
    out = _linear_ln(E.reshape(B * N * K, F), W, b, gain, bias)
    return out.reshape(B, N, K, W.shape[0]), edge_idx
